# baseline (speedup 1.0000x reference)
import functools

import jax
import jax.numpy as jnp
from jax import lax
from jax.experimental import pallas as pl
from jax.experimental.pallas import tpu as pltpu
from jax.experimental.pallas import tpu_sc as plsc

_K = 200
_W = 256
_WT = 1024
_RB = 8
_RB3 = 2
_MIN32 = -2147483648


def _sortkey(x):
    xi = lax.bitcast_convert_type(x + 0.0, jnp.int32)
    return xi ^ ((xi >> 31) & jnp.int32(0x7FFFFFFF))


def _slots_body(t_ref, slot_ref, pay_ref, nge_ref):
    n = t_ref.shape[1]
    ks = _sortkey(t_ref[...])
    tbits = jnp.zeros((_RB, 1), jnp.int32)
    for b in range(31, -1, -1):
        t2bits = tbits | jnp.int32((1 << b) - 4294967296 if b == 31 else 1 << b)
        t2s = t2bits ^ _MIN32
        cnt = jnp.sum((ks >= t2s).astype(jnp.int32), axis=1, keepdims=True)
        tbits = jnp.where(cnt >= _K, t2bits, tbits)
    kstar = tbits ^ _MIN32

    selb = ks >= kstar
    nge = jnp.sum(selb.astype(jnp.int32), axis=1, keepdims=True)
    nge_ref[...] = jnp.minimum(nge, _W) + jnp.zeros((_RB, _W), jnp.int32)

    pref = selb.astype(jnp.float32)
    sh = 1
    while sh < n:
        pref = pref + jnp.pad(pref, ((0, 0), (sh, 0)))[:, :n]
        sh *= 2
    pos = jnp.minimum(pref.astype(jnp.int32) - 1, _W - 1)
    pos = jnp.maximum(pos, 0)

    rowid = (lax.broadcasted_iota(jnp.int32, (_RB, n), 0)
             + pl.program_id(0) * _RB)
    col = lax.broadcasted_iota(jnp.int32, (_RB, n), 1)
    trash = _W + (col & (_WT - _W - 1))
    slot_ref[...] = rowid * _WT + jnp.where(selb, pos, trash)
    pay_ref[...] = rowid * n + col


def _sc_body(slot_hbm, pay_hbm, zero_hbm, tgtflat_hbm, logflat_hbm,
             cand_hbm, vout_hbm, sout_hbm,
             sl_v, pa_v, ix_v, vv_v, sv_v, sem):
    nhalf = sl_v.shape[0]
    npad = 2 * nhalf
    nrows = slot_hbm.shape[0] // npad
    rows_per_w = nrows // 32
    wid = lax.axis_index("s") * 2 + lax.axis_index("c")
    base_row = wid * rows_per_w

    for jj in range(rows_per_w):
        row = base_row + jj
        pltpu.sync_copy(zero_hbm, cand_hbm.at[pl.ds(row * _WT, 2 * 128)])
        for hh in range(2):
            off = row * npad + hh * nhalf
            pltpu.sync_copy(slot_hbm.at[pl.ds(off, nhalf)], sl_v)
            pltpu.sync_copy(pay_hbm.at[pl.ds(off, nhalf)], pa_v)
            pltpu.async_copy(pa_v, cand_hbm.at[sl_v], sem).wait()
        pltpu.sync_copy(cand_hbm.at[pl.ds(row * _WT, 128)], ix_v.at[0])
        pltpu.sync_copy(cand_hbm.at[pl.ds(row * _WT + 128, 128)], ix_v.at[1])
        cpa = pltpu.async_copy(tgtflat_hbm.at[ix_v.at[0]], vv_v.at[0], sem)
        cpb = pltpu.async_copy(tgtflat_hbm.at[ix_v.at[1]], vv_v.at[1], sem)
        cpc = pltpu.async_copy(logflat_hbm.at[ix_v.at[0]], sv_v.at[0], sem)
        cpd = pltpu.async_copy(logflat_hbm.at[ix_v.at[1]], sv_v.at[1], sem)
        cpa.wait()
        cpb.wait()
        cpc.wait()
        cpd.wait()
        pltpu.sync_copy(vv_v, vout_hbm.at[row])
        pltpu.sync_copy(sv_v, sout_hbm.at[row])


def _lb3(v_ref, s_ref, i_ref, g_ref, out_ref):
    W, K, RB = _W, _K, _RB3
    V = v_ref[0]; S = s_ref[0]; I = i_ref[0]
    col = lax.broadcasted_iota(jnp.int32, (RB, W), 1)
    slotv = col < g_ref[0]
    acc = jnp.sum(jnp.where(slotv, 1.0, 0.0), axis=1, keepdims=True)
    before = (slotv[:, None, :]
              & ((V[:, None, :] > V[:, :, None])
                 | ((V[:, None, :] == V[:, :, None])
                    & (I[:, None, :] < I[:, :, None]))))
    R = jnp.sum(before.astype(jnp.int32), axis=2)
    validR = jnp.where(slotv, R, K) < K
    acc = acc + jnp.sum(jnp.where(validR, 1.0, 0.0), axis=1, keepdims=True)
    m = jnp.max(jnp.where(validR, S, -jnp.inf), axis=1, keepdims=True)
    E = jnp.where(validR, jnp.exp(S - m), 0.0)
    acc = acc + m
    iot = lax.broadcasted_iota(jnp.int32, (RB, W, W), 1)
    U = jnp.sum(jnp.where(R[:, None, :] == iot, E[:, None, :], 0.0), axis=2)
    acc = acc + jnp.sum(U, axis=1, keepdims=True)
    r_i = lax.broadcasted_iota(jnp.int32, (W, W), 0)
    c_i = lax.broadcasted_iota(jnp.int32, (W, W), 1)
    tri = ((r_i >= c_i) & (r_i < K)).astype(jnp.float32)
    T = lax.dot_general(U, tri, (((1,), (0,)), ((), ())),
                        preferred_element_type=jnp.float32)
    acc = acc + jnp.sum(jnp.where(col < K, jnp.log(T), 0.0), axis=1,
                        keepdims=True)
    logT = jnp.where(col < K, jnp.log(T), 0.0)
    loss = (jnp.sum(logT, axis=1, keepdims=True)
            + K * m
            - jnp.sum(jnp.where(validR, S, 0.0), axis=1, keepdims=True))
    acc = loss
    out_ref[0] = acc + jnp.zeros((RB, 128), jnp.float32)


def kernel(logits, targets):
    b, n = logits.shape
    nblk = b // _RB
    npad = ((n + 1023) // 1024) * 1024

    slots, pay, nge2 = pl.pallas_call(
        _slots_body,
        grid=(nblk,),
        in_specs=[pl.BlockSpec((_RB, n), lambda i: (i, 0))],
        out_specs=[pl.BlockSpec((_RB, n), lambda i: (i, 0)),
                   pl.BlockSpec((_RB, n), lambda i: (i, 0)),
                   pl.BlockSpec((_RB, _W), lambda i: (i, 0))],
        out_shape=[jax.ShapeDtypeStruct((b, n), jnp.int32),
                   jax.ShapeDtypeStruct((b, n), jnp.int32),
                   jax.ShapeDtypeStruct((b, _W), jnp.int32)],
        compiler_params=pltpu.CompilerParams(
            dimension_semantics=("parallel",)),
    )(targets)

    padw = npad - n
    rows = jnp.arange(b, dtype=jnp.int32)[:, None]
    padslots = (rows * _WT + _W
                + (jnp.arange(padw, dtype=jnp.int32) & (_WT - _W - 1))[None, :]
                + jnp.zeros((b, padw), jnp.int32))
    padpay = rows * n + jnp.zeros((b, padw), jnp.int32)
    slots = jnp.concatenate([slots, padslots], axis=1)
    pay = jnp.concatenate([pay, padpay], axis=1)
    zero = jnp.zeros((2 * 128,), jnp.int32)
    mesh = plsc.VectorSubcoreMesh(core_axis_name="c", subcore_axis_name="s")
    sc_move = functools.partial(
        pl.kernel, mesh=mesh,
        out_type=[jax.ShapeDtypeStruct((b * _WT,), jnp.int32),
                  jax.ShapeDtypeStruct((b, 2, 128), jnp.float32),
                  jax.ShapeDtypeStruct((b, 2, 128), jnp.float32)],
        scratch_types=[pltpu.VMEM((npad // 2,), jnp.int32),
                       pltpu.VMEM((npad // 2,), jnp.int32),
                       pltpu.VMEM((2, 128), jnp.int32),
                       pltpu.VMEM((2, 128), jnp.float32),
                       pltpu.VMEM((2, 128), jnp.float32),
                       pltpu.SemaphoreType.DMA],
    )(_sc_body)
    cand, vv, sv = sc_move(slots.reshape(-1), pay.reshape(-1),
                           zero, targets.reshape(-1), logits.reshape(-1))
    vv = vv.reshape(b, _W)
    sv = sv.reshape(b, _W)
    ii = cand.reshape(b, _WT)[:, :_W] - (jnp.arange(b) * n)[:, None]

    nb3 = b // _RB3
    loss2 = pl.pallas_call(
        _lb3,
        grid=(nb3,),
        in_specs=[pl.BlockSpec((1, _RB3, _W), lambda i: (i, 0, 0)),
                  pl.BlockSpec((1, _RB3, _W), lambda i: (i, 0, 0)),
                  pl.BlockSpec((1, _RB3, _W), lambda i: (i, 0, 0)),
                  pl.BlockSpec((1, _RB3, _W), lambda i: (i, 0, 0))],
        out_specs=pl.BlockSpec((1, _RB3, 128), lambda i: (i, 0, 0)),
        out_shape=jax.ShapeDtypeStruct((nb3, _RB3, 128), jnp.float32),
    )(vv.reshape(nb3, _RB3, _W), sv.reshape(nb3, _RB3, _W),
      ii.reshape(nb3, _RB3, _W), nge2.reshape(nb3, _RB3, _W))
    loss2 = loss2.reshape(b, 128)

    return jnp.mean(loss2[:, 0]) / _K
